# baseline (speedup 1.0000x reference)
import functools

import jax
import jax.numpy as jnp
from jax import lax
from jax.experimental import pallas as pl
from jax.experimental.pallas import tpu as pltpu
from jax.experimental.pallas import tpu_sc as plsc

f32 = jnp.float32
i32 = jnp.int32

S = 128
TSIZE = 1 << 28
NPTS = 100000
NPAD = 102400
NC, NS = 2, 16
NW = NC * NS
CHUNK = NPAD // NW
JROWS = CHUNK // 128
BLK = 2048
GRID = NPAD // BLK
NPADE = NPAD + BLK
PADC = (127 * S + 127) * S * S + 127 * S + 127

DKS = [(dx * S + dy) * S + dz
       for dx in (-1, 0, 1) for dy in (-1, 0, 1) for dz in (-1, 0, 1)]

_mesh = plsc.VectorSubcoreMesh(core_axis_name="c", subcore_axis_name="s")


def _wid():
    return lax.axis_index("s") * NC + lax.axis_index("c")


def _iota16():
    return lax.iota(i32, 16)


@functools.partial(
    pl.kernel,
    out_type=(jax.ShapeDtypeStruct((NW, JROWS, 128), i32),
              jax.ShapeDtypeStruct((TSIZE,), i32)),
    mesh=_mesh,
    compiler_params=pltpu.CompilerParams(needs_layout_passes=False, use_tc_tiling_on_sc=False),
    scratch_types=[pltpu.VMEM((CHUNK * 4,), i32),
                   pltpu.VMEM((JROWS, 128), i32),
                   pltpu.VMEM((JROWS, 128), i32),
                   pltpu.SemaphoreType.DMA],
)
def _k1(coords_hbm, keys_out, t_out, cbuf, kidx, vals, sem):
    wid = _wid()
    base = wid * CHUNK
    pltpu.sync_copy(coords_hbm.at[pl.ds(base * 4, CHUNK * 4)], cbuf)

    def row(j, _):
        for c8 in range(8):
            off = c8 * 16
            ridx = j * 128 + off + _iota16()
            fidx = ridx * 4
            x0 = plsc.load_gather(cbuf, [fidx])
            x1 = plsc.load_gather(cbuf, [fidx + 1])
            x2 = plsc.load_gather(cbuf, [fidx + 2])
            x3 = plsc.load_gather(cbuf, [fidx + 3])
            k = ((x3 * S + x0) * S + x1) * S + x2
            kidx[j, pl.ds(off, 16)] = k
            vals[j, pl.ds(off, 16)] = base + ridx
        return 0

    lax.fori_loop(0, JROWS, row, 0)
    pltpu.sync_copy(kidx, keys_out.at[wid])
    for j in range(JROWS):
        pltpu.async_copy(vals.at[j], t_out.at[kidx.at[j]], sem)
    for j in range(JROWS):
        pltpu.make_async_copy(vals.at[j], t_out.at[kidx.at[j]], sem).wait()


SUB = 640
NSUB = CHUNK // SUB
GN = 27 * SUB // 128


@functools.partial(
    pl.kernel,
    out_type=jax.ShapeDtypeStruct((NW, CHUNK * 32), i32),
    mesh=_mesh,
    compiler_params=pltpu.CompilerParams(needs_layout_passes=False, use_tc_tiling_on_sc=False),
    scratch_types=[pltpu.VMEM((CHUNK,), i32),
                   pltpu.VMEM((27 * SUB,), i32),
                   pltpu.VMEM((27 * SUB,), i32),
                   pltpu.VMEM((27 * SUB,), i32),
                   pltpu.VMEM((SUB * 32,), i32),
                   pltpu.SemaphoreType.DMA,
                   pltpu.SemaphoreType.DMA],
)
def _k2(keys_hbm, t_hbm, nbr_out, kv, nkb, tb, vb, nbrbuf, sema, semb):
    wid = _wid()
    base = wid * CHUNK
    pltpu.sync_copy(keys_hbm.at[pl.ds(base, CHUNK)], kv)

    def sub(s, _):
        for k in range(27):
            dk = DKS[k]

            def g1(g, _, k=k, dk=dk):
                kvec = kv[pl.ds(s * SUB + g * 16, 16)]
                nkb[pl.ds(k * SUB + g * 16, 16)] = jnp.maximum(kvec + dk, 0)
                return 0

            lax.fori_loop(0, SUB // 16, g1, 0)

        def fire1(j, _):
            pltpu.async_copy(t_hbm.at[nkb.at[pl.ds(j * 128, 128)]],
                             tb.at[pl.ds(j * 128, 128)], sema)
            return 0

        lax.fori_loop(0, GN, fire1, 0)

        def drain1(j, _):
            pltpu.make_async_copy(t_hbm.at[nkb.at[pl.ds(j * 128, 128)]],
                                  tb.at[pl.ds(j * 128, 128)], sema).wait()
            return 0

        lax.fori_loop(0, GN, drain1, 0)

        def g2(g, _):
            t = tb[pl.ds(g * 16, 16)]
            tb[pl.ds(g * 16, 16)] = jnp.clip(t, 0, NPAD - 1)
            return 0

        lax.fori_loop(0, 27 * SUB // 16, g2, 0)

        def fire2(j, _):
            pltpu.async_copy(keys_hbm.at[tb.at[pl.ds(j * 128, 128)]],
                             vb.at[pl.ds(j * 128, 128)], semb)
            return 0

        lax.fori_loop(0, GN, fire2, 0)

        def drain2(j, _):
            pltpu.make_async_copy(keys_hbm.at[tb.at[pl.ds(j * 128, 128)]],
                                  vb.at[pl.ds(j * 128, 128)], semb).wait()
            return 0

        lax.fori_loop(0, GN, drain2, 0)

        for k in range(27):
            dk = DKS[k]

            def g4(g, _, k=k, dk=dk):
                tt = tb[pl.ds(k * SUB + g * 16, 16)]
                vk = vb[pl.ds(k * SUB + g * 16, 16)]
                nk0 = kv[pl.ds(s * SUB + g * 16, 16)] + dk
                sel = jnp.where(vk == nk0, tt, NPAD)
                plsc.store_scatter(nbrbuf, [(g * 16 + _iota16()) * 32 + k], sel)
                return 0

            lax.fori_loop(0, SUB // 16, g4, 0)
        for k in range(27, 32):

            def g5(g, _, k=k):
                plsc.store_scatter(nbrbuf, [(g * 16 + _iota16()) * 32 + k],
                                   jnp.full((16,), NPAD, i32))
                return 0

            lax.fori_loop(0, SUB // 16, g5, 0)
        pltpu.sync_copy(nbrbuf, nbr_out.at[wid, pl.ds(s * SUB * 32, SUB * 32)])
        return 0

    lax.fori_loop(0, NSUB, sub, 0)


@functools.partial(
    pl.kernel,
    out_type=jax.ShapeDtypeStruct((NPADE * 8,), f32),
    mesh=_mesh,
    compiler_params=pltpu.CompilerParams(needs_layout_passes=False, use_tc_tiling_on_sc=False),
    scratch_types=[pltpu.VMEM((CHUNK,), i32),
                   pltpu.VMEM((CHUNK * 4,), f32),
                   pltpu.VMEM((CHUNK * 8,), f32),
                   pltpu.VMEM((CHUNK,), f32)],
)
def _k2c(r_hbm, feats_hbm, zf_hbm, vox_out, rv, fbuf, acc, cacc):
    wid = _wid()
    lo = wid * CHUNK
    pltpu.sync_copy(zf_hbm, acc)
    pltpu.sync_copy(zf_hbm.at[pl.ds(0, CHUNK)], cacc)

    def chunk_scan(ch, _):
        pltpu.sync_copy(r_hbm.at[pl.ds(ch * CHUNK, CHUNK)], rv)
        pltpu.sync_copy(feats_hbm.at[pl.ds(ch * CHUNK * 4, CHUNK * 4)], fbuf)

        def grp(g, _):
            rr = rv[pl.ds(g * 16, 16)]
            m = (rr >= lo) & (rr < lo + CHUNK)
            local = jnp.clip(rr - lo, 0, CHUNK - 1)
            fq = (g * 16 + _iota16()) * 4
            for c in range(4):
                x = plsc.load_gather(fbuf, [fq + c])
                plsc.addupdate_scatter(acc, [local * 8 + c], x, mask=m)
            plsc.addupdate_scatter(cacc, [local], jnp.ones((16,), f32), mask=m)
            return 0

        lax.fori_loop(0, CHUNK // 16, grp, 0)
        return 0

    lax.fori_loop(0, NW, chunk_scan, 0)

    def fin(g, _):
        q = g * 16 + _iota16()
        cnt = plsc.load_gather(cacc, [lax.shift_right_logical(q, 3)])
        a = acc[pl.ds(g * 16, 16)]
        acc[pl.ds(g * 16, 16)] = a / jnp.maximum(cnt, 1.0)
        return 0

    lax.fori_loop(0, CHUNK * 8 // 16, fin, 0)
    pltpu.sync_copy(acc, vox_out.at[pl.ds(lo * 8, CHUNK * 8)])

    @pl.when(wid == NW - 1)
    def _():
        pltpu.sync_copy(zf_hbm.at[pl.ds(0, (NPADE - NPAD) * 8)],
                        vox_out.at[pl.ds(NPAD * 8, (NPADE - NPAD) * 8)])


def _make_gather(feat, subr):
    idxn = subr * 32
    nd = idxn // 128
    nsubg = CHUNK // subr

    @functools.partial(
        pl.kernel,
        out_type=jax.ShapeDtypeStruct((NW, CHUNK * 32, feat), f32),
        mesh=_mesh,
        compiler_params=pltpu.CompilerParams(needs_layout_passes=False, use_tc_tiling_on_sc=False),
        scratch_types=[pltpu.VMEM((idxn,), i32),
                       pltpu.VMEM((idxn, feat), f32),
                       pltpu.SemaphoreType.DMA],
    )
    def gk(nbr_hbm, tab_hbm, g_out, idxb, gbuf, sem):
        wid = _wid()
        base = wid * CHUNK * 32

        def subf(t, _):
            pltpu.sync_copy(nbr_hbm.at[pl.ds(base + t * idxn, idxn)], idxb)

            def fire(j, _):
                pltpu.async_copy(tab_hbm.at[idxb.at[pl.ds(j * 128, 128)]],
                                 gbuf.at[pl.ds(j * 128, 128), :], sem)
                return 0

            lax.fori_loop(0, nd, fire, 0)

            def drain(j, _):
                pltpu.make_async_copy(tab_hbm.at[idxb.at[pl.ds(j * 128, 128)]],
                                      gbuf.at[pl.ds(j * 128, 128), :], sem).wait()
                return 0

            lax.fori_loop(0, nd, drain, 0)
            pltpu.sync_copy(gbuf, g_out.at[wid, pl.ds(t * idxn, idxn), :])
            return 0

        lax.fori_loop(0, nsubg, subf, 0)

    return gk


_k3 = _make_gather(8, 320)
_k7 = _make_gather(32, 64)


SUBO = 640


@functools.partial(
    pl.kernel,
    out_type=jax.ShapeDtypeStruct((NW, CHUNK, 32), f32),
    mesh=_mesh,
    compiler_params=pltpu.CompilerParams(needs_layout_passes=False, use_tc_tiling_on_sc=False),
    scratch_types=[pltpu.VMEM((SUBO,), i32),
                   pltpu.VMEM((SUBO, 32), f32),
                   pltpu.SemaphoreType.DMA],
)
def _k10(r_hbm, logits_hbm, out_hbm, idxb, gbuf, sem):
    wid = _wid()
    base = wid * CHUNK

    def subf(t, _):
        pltpu.sync_copy(r_hbm.at[pl.ds(base + t * SUBO, SUBO)], idxb)

        def fire(j, _):
            pltpu.async_copy(logits_hbm.at[idxb.at[pl.ds(j * 128, 128)]],
                             gbuf.at[pl.ds(j * 128, 128), :], sem)
            return 0

        lax.fori_loop(0, SUBO // 128, fire, 0)

        def drain(j, _):
            pltpu.make_async_copy(logits_hbm.at[idxb.at[pl.ds(j * 128, 128)]],
                                  gbuf.at[pl.ds(j * 128, 128), :], sem).wait()
            return 0

        lax.fori_loop(0, SUBO // 128, drain, 0)
        pltpu.sync_copy(gbuf, out_hbm.at[wid, pl.ds(t * SUBO, SUBO), :])
        return 0

    lax.fori_loop(0, CHUNK // SUBO, subf, 0)


def _make_mm_stats(kdim, blkr):
    grid = NPAD // blkr

    def body(g_ref, w_ref, r_ref, h_ref, st_ref, nv_ref, sacc, qacc, nvacc):
        b = pl.program_id(0)

        @pl.when(b == 0)
        def _():
            sacc[...] = jnp.zeros_like(sacc)
            qacc[...] = jnp.zeros_like(qacc)
            nvacc[...] = jnp.zeros_like(nvacc)

        h = jnp.dot(g_ref[...], w_ref[...], preferred_element_type=f32,
                    precision=lax.Precision.HIGHEST)
        row = b * blkr + lax.broadcasted_iota(i32, (blkr, 1), 0)
        vf = ((r_ref[...] == row) & (row < NPTS)).astype(f32)
        hm = h * vf
        h_ref[...] = h
        sacc[...] += jnp.sum(hm, axis=0, keepdims=True)
        qacc[...] += jnp.sum(hm * h, axis=0, keepdims=True)
        nvacc[...] += jnp.sum(vf).reshape(1, 1)

        @pl.when(b == grid - 1)
        def _():
            st_ref[0:1, :] = sacc[...]
            st_ref[1:2, :] = qacc[...]
            nv_ref[...] = jnp.maximum(nvacc[...], 1.0)

    return pl.pallas_call(
        body,
        grid=(grid,),
        in_specs=[pl.BlockSpec((blkr, kdim), lambda b: (b, 0)),
                  pl.BlockSpec((kdim, 32), lambda b: (0, 0)),
                  pl.BlockSpec((blkr, 1), lambda b: (b, 0))],
        out_specs=[pl.BlockSpec((blkr, 32), lambda b: (b, 0)),
                   pl.BlockSpec((2, 32), lambda b: (0, 0)),
                   pl.BlockSpec((1, 1), lambda b: (0, 0))],
        out_shape=[jax.ShapeDtypeStruct((NPAD, 32), f32),
                   jax.ShapeDtypeStruct((2, 32), f32),
                   jax.ShapeDtypeStruct((1, 1), f32)],
        scratch_shapes=[pltpu.VMEM((1, 32), f32),
                        pltpu.VMEM((1, 32), f32),
                        pltpu.VMEM((1, 1), f32)],
    )


_k5 = _make_mm_stats(256, BLK)
_k8 = _make_mm_stats(1024, 512)


def _k6_body(h_ref, st_ref, nv_ref, g_ref, be_ref, out_ref):
    b = pl.program_id(0)
    nv = nv_ref[0, 0]
    m = st_ref[0:1, :] / nv
    var = st_ref[1:2, :] / nv - m * m
    y = (h_ref[...] - m) * lax.rsqrt(var + 1e-5) * g_ref[...] + be_ref[...]
    y = jnp.maximum(y, 0.0)
    row = b * BLK + lax.broadcasted_iota(i32, (BLK, 1), 0)
    out_ref[...] = jnp.where(row < NPAD, y, 0.0)


_k6 = pl.pallas_call(
    _k6_body,
    grid=(NPADE // BLK,),
    in_specs=[pl.BlockSpec((BLK, 32), lambda b: (jnp.minimum(b, GRID - 1), 0)),
              pl.BlockSpec((2, 32), lambda b: (0, 0)),
              pl.BlockSpec((1, 1), lambda b: (0, 0)),
              pl.BlockSpec((1, 32), lambda b: (0, 0)),
              pl.BlockSpec((1, 32), lambda b: (0, 0))],
    out_specs=pl.BlockSpec((BLK, 32), lambda b: (b, 0)),
    out_shape=jax.ShapeDtypeStruct((NPADE, 32), f32),
)


def _k9_body(h_ref, st_ref, nv_ref, g_ref, be_ref, wc_ref, bc_ref, out_ref):
    nv = nv_ref[0, 0]
    m = st_ref[0:1, :] / nv
    var = st_ref[1:2, :] / nv - m * m
    y = (h_ref[...] - m) * lax.rsqrt(var + 1e-5) * g_ref[...] + be_ref[...]
    y = jnp.maximum(y, 0.0)
    out_ref[...] = jnp.dot(y, wc_ref[...], preferred_element_type=f32,
                    precision=lax.Precision.HIGHEST) + bc_ref[...]


_k9 = pl.pallas_call(
    _k9_body,
    grid=(GRID,),
    in_specs=[pl.BlockSpec((BLK, 32), lambda b: (b, 0)),
              pl.BlockSpec((2, 32), lambda b: (0, 0)),
              pl.BlockSpec((1, 1), lambda b: (0, 0)),
              pl.BlockSpec((1, 32), lambda b: (0, 0)),
              pl.BlockSpec((1, 32), lambda b: (0, 0)),
              pl.BlockSpec((32, 32), lambda b: (0, 0)),
              pl.BlockSpec((1, 32), lambda b: (0, 0))],
    out_specs=pl.BlockSpec((BLK, 32), lambda b: (b, 0)),
    out_shape=jax.ShapeDtypeStruct((NPAD, 32), f32),
)


def kernel(feats, coords, W1, W2, g1, be1, g2, be2, Wc, bc):
    coords_p = jnp.concatenate(
        [coords.astype(i32), jnp.full((NPAD - NPTS, 4), 127, i32)], axis=0)
    feats_p = jnp.concatenate(
        [feats, jnp.zeros((NPAD - NPTS, 4), f32)], axis=0)

    keys3, table = _k1(coords_p.reshape(NPAD * 4))
    keys = keys3.reshape(NPAD)
    nbr3 = _k2(keys, table)
    nbr_flat = nbr3.reshape(NPAD * 32)
    r = nbr3.reshape(NPAD, 32)[:, 13]
    r_col = r.reshape(NPAD, 1)

    vox_ext = _k2c(r, feats_p.reshape(NPAD * 4),
                   jnp.zeros((CHUNK * 8,), f32)).reshape(NPADE, 8)

    g1mat = _k3(nbr_flat, vox_ext).reshape(NPAD, 256)
    w1p = jnp.concatenate(
        [jnp.concatenate([W1, jnp.zeros((27, 4, 32), f32)], 1).reshape(216, 32),
         jnp.zeros((40, 32), f32)], 0)
    h1, st1, nv1 = _k5(g1mat, w1p, r_col)
    h1_ext = _k6(h1, st1, nv1, g1.reshape(1, 32), be1.reshape(1, 32))

    g2mat = _k7(nbr_flat, h1_ext).reshape(NPAD, 1024)
    w2p = jnp.concatenate([W2.reshape(864, 32), jnp.zeros((160, 32), f32)], 0)
    h2, st2, nv2 = _k8(g2mat, w2p, r_col)

    wcp = jnp.concatenate([Wc, jnp.zeros((32, 12), f32)], 1)
    bcp = jnp.concatenate([bc, jnp.zeros((12,), f32)]).reshape(1, 32)
    logits = _k9(h2, st2, nv2, g2.reshape(1, 32), be2.reshape(1, 32), wcp, bcp)

    outg = _k10(r, logits).reshape(NPAD, 32)
    return outg[:NPTS, :20]
